# Initial kernel scaffold; baseline (speedup 1.0000x reference)
import jax
import jax.numpy as jnp
from jax.experimental import pallas as pl


def kernel(x_word, x_bigram, x_trigram, emb_word, emb_bigram, emb_trigram, W1, b1, W2, b2):
    raise NotImplementedError("write your pallas kernel here")


# speedup vs baseline: 2.5217x; 2.5217x over previous
import functools

import jax
import jax.numpy as jnp
from jax import lax
from jax.experimental import pallas as pl
from jax.experimental.pallas import tpu as pltpu
from jax.experimental.pallas import tpu_sc as plsc

B = 4096
L = 200
EMB = 64
HID = 256
NCLS = 10
POOL_W = 3 * EMB

_NC = 2
_NS = 16
_NW = _NC * _NS
_ROWS_PER_W = B // _NW
_CHUNKS = ((0, 128), (128, 72))


def _sc_pool(x_word, x_bigram, x_trigram, emb_word, emb_bigram, emb_trigram):
    mesh = plsc.VectorSubcoreMesh(core_axis_name="c", subcore_axis_name="s")

    @functools.partial(
        pl.kernel,
        mesh=mesh,
        compiler_params=pltpu.CompilerParams(use_tc_tiling_on_sc=False),
        out_type=jax.ShapeDtypeStruct((B, POOL_W), jnp.float32),
        scratch_types=[
            pltpu.VMEM((_ROWS_PER_W, L), jnp.int32),
            pltpu.VMEM((L, EMB), jnp.float32),
            pltpu.VMEM((_ROWS_PER_W, POOL_W), jnp.float32),
            pltpu.SemaphoreType.DMA,
        ],
    )
    def pool_kernel(xw, xb, xt, ew, eb, et, out, idx_v, rows_v, out_v, sem):
        wid = lax.axis_index("s") * _NC + lax.axis_index("c")
        base = wid * _ROWS_PER_W

        for t, (x_hbm, tab_hbm) in enumerate(((xw, ew), (xb, eb), (xt, et))):
            pltpu.sync_copy(x_hbm.at[pl.ds(base, _ROWS_PER_W), :], idx_v)

            def row_body(i, _, tab_hbm=tab_hbm, t=t):
                cps = [
                    pltpu.async_copy(
                        tab_hbm.at[idx_v.at[i, pl.ds(off, sz)]],
                        rows_v.at[pl.ds(off, sz), :],
                        sem,
                    )
                    for off, sz in _CHUNKS
                ]
                for cp in cps:
                    cp.wait()

                def acc_body(r, accs):
                    return tuple(
                        a + rows_v[r, pl.ds(16 * c, 16)]
                        for c, a in enumerate(accs)
                    )

                z = jnp.zeros((16,), jnp.float32)
                accs = lax.fori_loop(0, L, acc_body, (z, z, z, z))
                for c in range(4):
                    out_v[i, pl.ds(t * EMB + 16 * c, 16)] = accs[c]
                return 0

            lax.fori_loop(0, _ROWS_PER_W, row_body, 0)

        pltpu.sync_copy(out_v, out.at[pl.ds(base, _ROWS_PER_W), :])

    return pool_kernel(x_word, x_bigram, x_trigram,
                       emb_word, emb_bigram, emb_trigram)


def _mlp_body(x_ref, w1_ref, b1_ref, w2_ref, b2_ref, o_ref):
    h = jnp.dot(x_ref[...], w1_ref[...], preferred_element_type=jnp.float32)
    h = h * (1.0 / L) + b1_ref[...]
    h = jnp.maximum(h, 0.0)
    o = jnp.dot(h, w2_ref[...], preferred_element_type=jnp.float32)
    o_ref[...] = o + b2_ref[...]


def _tc_mlp(pooled, W1, b1, W2, b2):
    blk = 512
    grid = (B // blk,)
    return pl.pallas_call(
        _mlp_body,
        grid=grid,
        in_specs=[
            pl.BlockSpec((blk, POOL_W), lambda i: (i, 0)),
            pl.BlockSpec((POOL_W, HID), lambda i: (0, 0)),
            pl.BlockSpec((1, HID), lambda i: (0, 0)),
            pl.BlockSpec((HID, NCLS), lambda i: (0, 0)),
            pl.BlockSpec((1, NCLS), lambda i: (0, 0)),
        ],
        out_specs=pl.BlockSpec((blk, NCLS), lambda i: (i, 0)),
        out_shape=jax.ShapeDtypeStruct((B, NCLS), jnp.float32),
    )(pooled, W1, b1.reshape(1, HID), W2, b2.reshape(1, NCLS))


def kernel(x_word, x_bigram, x_trigram, emb_word, emb_bigram, emb_trigram,
           W1, b1, W2, b2):
    pooled = _sc_pool(x_word, x_bigram, x_trigram,
                      emb_word, emb_bigram, emb_trigram)
    return _tc_mlp(pooled, W1, b1, W2, b2)
